# baseline (speedup 1.0000x reference)
import functools

import jax
import jax.numpy as jnp
from jax import lax
from jax.experimental import pallas as pl
from jax.experimental.pallas import tpu as pltpu
from jax.experimental.pallas import tpu_sc as plsc

_NC, _NS, _L = 2, 16, 16


def _tc_copy_body(kb_hbm, vb_hbm, out_hbm, sem, *, m, chunks):
    rows_per = m // chunks
    copies = []
    for src, off in ((kb_hbm, 0), (vb_hbm, m)):
        for c in range(chunks):
            b = c * rows_per
            cp = pltpu.make_async_copy(
                src.at[pl.ds(b, rows_per)],
                out_hbm.at[pl.ds(off + b, rows_per)],
                sem,
            )
            cp.start()
            copies.append(cp)
    for cp in copies:
        cp.wait()


def _sc_scatter_body(k_hbm, v_hbm, idx_hbm, out_hbm, idx_v, *, m, nb, rows):
    wid = lax.axis_index("s") * _NC + lax.axis_index("c")
    base = wid * rows
    pltpu.sync_copy(idx_hbm, idx_v)
    for c in range(nb // _L):
        chunk = idx_v[pl.ds(c * _L, _L)]
        for j in range(_L):
            b = c * _L + j
            t = chunk[j]

            @pl.when((t >= base) & (t < base + rows))
            def _():
                pltpu.sync_copy(k_hbm.at[pl.ds(b, 1)], out_hbm.at[pl.ds(t, 1)])
                pltpu.sync_copy(v_hbm.at[pl.ds(b, 1)],
                                out_hbm.at[pl.ds(m + t, 1)])


def kernel(k, v, mem_index, key_buffer, value_buffer):
    m, h, d = key_buffer.shape
    nb = k.shape[0]
    hd = h * d

    out_ref = jax.empty_ref(jax.ShapeDtypeStruct((2 * m, hd), key_buffer.dtype))

    tc_mesh = pltpu.create_tensorcore_mesh("x", num_cores=1)
    copy_body = functools.partial(_tc_copy_body, m=m, chunks=8)
    pl.kernel(
        copy_body,
        out_type=(),
        mesh=tc_mesh,
        scratch_types=[pltpu.SemaphoreType.DMA],
    )(key_buffer.reshape(m, hd), value_buffer.reshape(m, hd), out_ref)

    rows = m // (_NC * _NS)
    scatter_body = functools.partial(_sc_scatter_body, m=m, nb=nb, rows=rows)
    sc_mesh = plsc.VectorSubcoreMesh(core_axis_name="c", subcore_axis_name="s")
    pl.kernel(
        scatter_body,
        out_type=(),
        mesh=sc_mesh,
        scratch_types=[pltpu.VMEM((nb,), jnp.int32)],
    )(k.reshape(nb, hd), v.reshape(nb, hd), mem_index.astype(jnp.int32),
      out_ref)

    return jax.freeze(out_ref).reshape(2, m, h, d)


# speedup vs baseline: 14.5663x; 14.5663x over previous
import functools

import jax
import jax.numpy as jnp
from jax import lax
from jax.experimental import pallas as pl
from jax.experimental.pallas import tpu as pltpu
from jax.experimental.pallas import tpu_sc as plsc

_NC, _NS, _L = 2, 16, 16


def _tc_copy_body(kb_hbm, vb_hbm, out_hbm, *, m, hd, bm):
    nblk = m // bm

    def inner(src_ref, dst_ref):
        dst_ref[...] = src_ref[...]

    for src, blk_off in ((kb_hbm, 0), (vb_hbm, nblk)):
        pltpu.emit_pipeline(
            inner,
            grid=(nblk,),
            in_specs=[pl.BlockSpec((bm, hd), lambda i: (i, 0))],
            out_specs=[
                pl.BlockSpec((bm, hd),
                             lambda i, _o=blk_off: (i + _o, 0))
            ],
        )(src, out_hbm)


def _sc_scatter_body(k_hbm, v_hbm, idx_hbm, out_hbm, idx_v, *, m, nb, rows):
    wid = lax.axis_index("s") * _NC + lax.axis_index("c")
    base = wid * rows
    pltpu.sync_copy(idx_hbm, idx_v)
    for c in range(nb // _L):
        chunk = idx_v[pl.ds(c * _L, _L)]
        for j in range(_L):
            b = c * _L + j
            t = chunk[j]

            @pl.when((t >= base) & (t < base + rows))
            def _():
                pltpu.sync_copy(k_hbm.at[pl.ds(b, 1)], out_hbm.at[pl.ds(t, 1)])
                pltpu.sync_copy(v_hbm.at[pl.ds(b, 1)],
                                out_hbm.at[pl.ds(m + t, 1)])


def kernel(k, v, mem_index, key_buffer, value_buffer):
    m, h, d = key_buffer.shape
    nb = k.shape[0]
    hd = h * d

    out_ref = jax.empty_ref(jax.ShapeDtypeStruct((2 * m, hd), key_buffer.dtype))

    tc_mesh = pltpu.create_tensorcore_mesh("x", num_cores=1)
    copy_body = functools.partial(_tc_copy_body, m=m, hd=hd, bm=1024)
    pl.kernel(
        copy_body,
        out_type=(),
        mesh=tc_mesh,
    )(key_buffer.reshape(m, hd), value_buffer.reshape(m, hd), out_ref)

    rows = m // (_NC * _NS)
    scatter_body = functools.partial(_sc_scatter_body, m=m, nb=nb, rows=rows)
    sc_mesh = plsc.VectorSubcoreMesh(core_axis_name="c", subcore_axis_name="s")
    pl.kernel(
        scatter_body,
        out_type=(),
        mesh=sc_mesh,
        scratch_types=[pltpu.VMEM((nb,), jnp.int32)],
    )(k.reshape(nb, hd), v.reshape(nb, hd), mem_index.astype(jnp.int32),
      out_ref)

    return jax.freeze(out_ref).reshape(2, m, h, d)


# speedup vs baseline: 37.8995x; 2.6019x over previous
import functools

import jax
import jax.numpy as jnp
from jax import lax
from jax.experimental import pallas as pl
from jax.experimental.pallas import tpu as pltpu
from jax.experimental.pallas import tpu_sc as plsc

_NC, _NS, _L = 2, 16, 16


def _tc_copy_body(kb_hbm, vb_hbm, out_hbm, *, m, h, d, bm):
    nblk = m // bm

    def inner(src_ref, dst_ref):
        dst_ref[0] = src_ref[...]

    for src, half in ((kb_hbm, 0), (vb_hbm, 1)):
        pltpu.emit_pipeline(
            inner,
            grid=(nblk,),
            in_specs=[pl.BlockSpec((bm, h, d), lambda i: (i, 0, 0))],
            out_specs=[
                pl.BlockSpec((1, bm, h, d),
                             lambda i, _s=half: (_s, i, 0, 0))
            ],
        )(src, out_hbm)


def _sc_scatter_body(k_hbm, v_hbm, idx_hbm, out_hbm, idx_v, *, nb, rows):
    wid = lax.axis_index("s") * _NC + lax.axis_index("c")
    base = wid * rows
    pltpu.sync_copy(idx_hbm, idx_v)
    for c in range(nb // _L):
        chunk = idx_v[pl.ds(c * _L, _L)]
        for j in range(_L):
            b = c * _L + j
            t = chunk[j]

            @pl.when((t >= base) & (t < base + rows))
            def _():
                pltpu.sync_copy(k_hbm.at[pl.ds(b, 1)],
                                out_hbm.at[0, pl.ds(t, 1)])
                pltpu.sync_copy(v_hbm.at[pl.ds(b, 1)],
                                out_hbm.at[1, pl.ds(t, 1)])


def kernel(k, v, mem_index, key_buffer, value_buffer):
    m, h, d = key_buffer.shape
    nb = k.shape[0]

    out_ref = jax.empty_ref(
        jax.ShapeDtypeStruct((2, m, h, d), key_buffer.dtype))

    tc_mesh = pltpu.create_tensorcore_mesh("x", num_cores=1)
    copy_body = functools.partial(_tc_copy_body, m=m, h=h, d=d, bm=1024)
    pl.kernel(
        copy_body,
        out_type=(),
        mesh=tc_mesh,
    )(key_buffer, value_buffer, out_ref)

    rows = m // (_NC * _NS)
    scatter_body = functools.partial(_sc_scatter_body, nb=nb, rows=rows)
    sc_mesh = plsc.VectorSubcoreMesh(core_axis_name="c", subcore_axis_name="s")
    pl.kernel(
        scatter_body,
        out_type=(),
        mesh=sc_mesh,
        scratch_types=[pltpu.VMEM((nb,), jnp.int32)],
    )(k.reshape(nb, h, d), v.reshape(nb, h, d),
      mem_index.astype(jnp.int32), out_ref)

    return jax.freeze(out_ref)


# speedup vs baseline: 38.0153x; 1.0031x over previous
import functools

import jax
import jax.numpy as jnp
from jax import lax
from jax.experimental import pallas as pl
from jax.experimental.pallas import tpu as pltpu
from jax.experimental.pallas import tpu_sc as plsc

_NC, _NS, _L = 2, 16, 16


def _tc_copy_body(kb_hbm, vb_hbm, out_hbm, *, m, h, d, bm):
    nblk = m // bm

    def inner(src_ref, dst_ref):
        dst_ref[0] = src_ref[...]

    for src, half in ((kb_hbm, 0), (vb_hbm, 1)):
        pltpu.emit_pipeline(
            inner,
            grid=(nblk,),
            in_specs=[pl.BlockSpec((bm, h, d), lambda i: (i, 0, 0))],
            out_specs=[
                pl.BlockSpec((1, bm, h, d),
                             lambda i, _s=half: (_s, i, 0, 0))
            ],
        )(src, out_hbm)


def _sc_scatter_body(k_hbm, v_hbm, idx_hbm, out_hbm, idx_v, sem, *, nb, rows):
    wid = lax.axis_index("s") * _NC + lax.axis_index("c")
    base = wid * rows
    pltpu.sync_copy(idx_hbm, idx_v)
    nchunk = nb // _L
    iota = lax.iota(jnp.int32, _L)
    chunks = [idx_v[pl.ds(c * _L, _L)] for c in range(nchunk)]

    oks, ts = [], []
    for c in range(nchunk):
        for j in range(_L):
            t = chunks[c][j]
            tv = jnp.full((_L,), t, dtype=jnp.int32)
            later = jnp.int32(0)
            for c2 in range(c, nchunk):
                eq = chunks[c2] == tv
                if c2 == c:
                    eq = eq & (iota > j)
                later = later + plsc.all_reduce_population_count(eq)[0]
            oks.append((t >= base) & (t < base + rows) & (later == 0))
            ts.append(t)

    def _copies(b):
        t = ts[b]
        kcp = pltpu.make_async_copy(k_hbm.at[pl.ds(b, 1)],
                                    out_hbm.at[0, pl.ds(t, 1)], sem)
        vcp = pltpu.make_async_copy(v_hbm.at[pl.ds(b, 1)],
                                    out_hbm.at[1, pl.ds(t, 1)], sem)
        return kcp, vcp

    for b in range(nb):
        @pl.when(oks[b])
        def _():
            kcp, vcp = _copies(b)
            kcp.start()
            vcp.start()

    for b in range(nb):
        @pl.when(oks[b])
        def _():
            kcp, vcp = _copies(b)
            kcp.wait()
            vcp.wait()


def kernel(k, v, mem_index, key_buffer, value_buffer):
    m, h, d = key_buffer.shape
    nb = k.shape[0]

    out_ref = jax.empty_ref(
        jax.ShapeDtypeStruct((2, m, h, d), key_buffer.dtype))

    tc_mesh = pltpu.create_tensorcore_mesh("x", num_cores=1)
    copy_body = functools.partial(_tc_copy_body, m=m, h=h, d=d, bm=1024)
    pl.kernel(
        copy_body,
        out_type=(),
        mesh=tc_mesh,
    )(key_buffer, value_buffer, out_ref)

    rows = m // (_NC * _NS)
    scatter_body = functools.partial(_sc_scatter_body, nb=nb, rows=rows)
    sc_mesh = plsc.VectorSubcoreMesh(core_axis_name="c", subcore_axis_name="s")
    pl.kernel(
        scatter_body,
        out_type=(),
        mesh=sc_mesh,
        compiler_params=pltpu.CompilerParams(needs_layout_passes=False),
        scratch_types=[pltpu.VMEM((nb,), jnp.int32),
                       pltpu.SemaphoreType.DMA],
    )(k.reshape(nb, h, d), v.reshape(nb, h, d),
      mem_index.astype(jnp.int32), out_ref)

    return jax.freeze(out_ref)
